# baseline (speedup 1.0000x reference)
import functools

import jax
import jax.numpy as jnp
from jax import lax
from jax.experimental import pallas as pl
from jax.experimental.pallas import tpu as pltpu
from jax.experimental.pallas import tpu_sc as plsc

N = 10000
D = 128
DH = D // 2
E = 320000

NC = 2
NS = 16
CHUNK = 64
C_PER_T = 320
B_IDX = 20
NBLK = C_PER_T // B_IDX
E_PAD = NS * C_PER_T * CHUNK
N_PAD = 10240
ROWS_PER_TILE = N_PAD // NS
WCHUNKS = ROWS_PER_TILE // CHUNK
NBUF = 4
CROWS = 632


def _sc_segment_sum(xas, srcs, dsts, zrows):
    mesh = plsc.VectorSubcoreMesh(
        core_axis_name="c", subcore_axis_name="s", num_cores=NC, num_subcores=NS
    )

    @functools.partial(
        pl.kernel,
        out_type=jax.ShapeDtypeStruct((NC, N_PAD, DH), jnp.float32),
        mesh=mesh,
        compiler_params=pltpu.CompilerParams(use_tc_tiling_on_sc=False),
        scratch_types=[
            pltpu.VMEM((B_IDX, CHUNK), jnp.int32),
            pltpu.VMEM((B_IDX, CHUNK), jnp.int32),
            [pltpu.VMEM((CHUNK, DH), jnp.float32) for _ in range(NBUF)],
            pltpu.MemorySpace.VMEM_SHARED((N, DH), jnp.float32),
            pltpu.MemorySpace.VMEM_SHARED((N_PAD, DH), jnp.float32),
            [pltpu.SemaphoreType.DMA for _ in range(NBUF)],
            [pltpu.SemaphoreType.DMA for _ in range(NBUF)],
        ],
    )
    def sc_kernel(xa_hbm, src_hbm, dst_hbm, z_hbm, out_hbm,
                  idx_s, idx_d, rows, x_cache, agg_sh, gsem, ssem):
        c = lax.axis_index("c")
        s = lax.axis_index("s")

        @pl.when(s < NS - 1)
        def _():
            pltpu.sync_copy(xa_hbm.at[c, pl.ds(s * CROWS, CROWS)],
                            x_cache.at[pl.ds(s * CROWS, CROWS)])

        @pl.when(s == NS - 1)
        def _():
            pltpu.sync_copy(xa_hbm.at[c, pl.ds((NS - 1) * CROWS, N - (NS - 1) * CROWS)],
                            x_cache.at[pl.ds((NS - 1) * CROWS, N - (NS - 1) * CROWS)])

        pltpu.sync_copy(z_hbm, rows[0])
        for k in range(WCHUNKS):
            pltpu.sync_copy(rows[0], agg_sh.at[pl.ds(s * ROWS_PER_TILE + k * CHUNK, CHUNK)])
        plsc.subcore_barrier()

        def blk_body(gblk, carry):
            pltpu.sync_copy(src_hbm.at[s, gblk], idx_s)
            pltpu.sync_copy(dst_hbm.at[s, gblk], idx_d)

            def body(g, carry2):
                base = g * NBUF
                descs = []
                for r in range(NBUF):
                    descs.append(pltpu.async_copy(
                        x_cache.at[idx_s.at[base + r]], rows[r], gsem[r]))
                sdescs = []
                for r in range(NBUF):
                    descs[r].wait()
                    sdescs.append(pltpu.async_copy(
                        rows[r], agg_sh.at[idx_d.at[base + r]], ssem[r], add=True))
                for r in range(NBUF):
                    sdescs[r].wait()
                return carry2

            lax.fori_loop(0, B_IDX // NBUF, body, 0)
            return carry

        lax.fori_loop(0, NBLK, blk_body, 0)
        plsc.subcore_barrier()

        for k in range(WCHUNKS):
            off = s * ROWS_PER_TILE + k * CHUNK
            pltpu.sync_copy(agg_sh.at[pl.ds(off, CHUNK)], rows[k % NBUF])
            pltpu.sync_copy(rows[k % NBUF], out_hbm.at[c, pl.ds(off, CHUNK)])

    return sc_kernel(xas, srcs, dsts, zrows)


BLK = 1000


def _tc_body(p_ref, xa_ref, wl_ref, wc_ref, wf_ref, b1_ref, bf_ref, o_ref):
    xa = xa_ref[...]
    wl = wl_ref[...]
    t = (jnp.dot(p_ref[0], wl[:DH, :], preferred_element_type=jnp.float32)
         + jnp.dot(p_ref[1], wl[DH:, :], preferred_element_type=jnp.float32)
         + jnp.dot(xa, wc_ref[...], preferred_element_type=jnp.float32)
         + b1_ref[...])
    o_ref[...] = (jnp.dot(jnp.maximum(t, 0.0), wf_ref[...],
                          preferred_element_type=jnp.float32)
                  + bf_ref[...])


def _tc_fused(partials, x_a, wl_t, wc_t, wf_t, b1, bf):
    return pl.pallas_call(
        _tc_body,
        grid=(N // BLK,),
        in_specs=[
            pl.BlockSpec((NC, BLK, DH), lambda j: (0, j, 0)),
            pl.BlockSpec((BLK, D), lambda j: (j, 0)),
            pl.BlockSpec((D, D), lambda j: (0, 0)),
            pl.BlockSpec((D, D), lambda j: (0, 0)),
            pl.BlockSpec((D, D), lambda j: (0, 0)),
            pl.BlockSpec((1, D), lambda j: (0, 0)),
            pl.BlockSpec((1, D), lambda j: (0, 0)),
        ],
        out_specs=pl.BlockSpec((BLK, D), lambda j: (j, 0)),
        out_shape=jax.ShapeDtypeStruct((N, D), jnp.float32),
    )(partials, x_a, wl_t, wc_t, wf_t, b1, bf)


def kernel(x_A, x_B, edge_r0, edge_r1,
           Wl0, bl0, W00, b00, W10, b10,
           Wl1, bl1, W01, b01, W11, b11,
           Wm0, bm0, Wm1, bm1, Wout, bout):
    src = edge_r1[1]
    dst = edge_r1[0]
    pad = E_PAD - E
    srcs = jnp.concatenate([src, jnp.zeros((pad,), jnp.int32)])
    pad_dst = N + (jnp.arange(pad, dtype=jnp.int32) % (N_PAD - N))
    dsts = jnp.concatenate([dst, pad_dst])
    srcs = srcs.reshape(NS, NBLK, B_IDX, CHUNK)
    dsts = dsts.reshape(NS, NBLK, B_IDX, CHUNK)
    zrows = jnp.zeros((CHUNK, DH), jnp.float32)
    xas = jnp.stack([x_A[:, :DH], x_A[:, DH:]], axis=0)

    partials = _sc_segment_sum(xas, srcs, dsts, zrows)

    eye = jnp.eye(D, dtype=jnp.float32)
    wl_t = Wl1.T
    wc_t = (W01 + W11).T + eye
    b1 = (bl1 + b01 + b11).reshape(1, D)
    wf_t = (Wout @ Wm1).T
    bf = (bm1 @ Wout.T + bout).reshape(1, D)

    return _tc_fused(partials, x_A, wl_t, wc_t, wf_t, b1, bf)


# speedup vs baseline: 1.0277x; 1.0277x over previous
import functools

import jax
import jax.numpy as jnp
from jax import lax
from jax.experimental import pallas as pl
from jax.experimental.pallas import tpu as pltpu
from jax.experimental.pallas import tpu_sc as plsc

N = 10000
D = 128
DH = D // 2
E = 320000

NC = 2
NS = 16
CHUNK = 128
C_PER_T = 160
B_IDX = 20
NBLK = C_PER_T // B_IDX
E_PAD = NS * C_PER_T * CHUNK
N_PAD = 10240
ROWS_PER_TILE = N_PAD // NS
WCHUNKS = ROWS_PER_TILE // CHUNK
NBUF = 4
CROWS = 632


def _sc_segment_sum(xas, srcs, dsts, zrows):
    mesh = plsc.VectorSubcoreMesh(
        core_axis_name="c", subcore_axis_name="s", num_cores=NC, num_subcores=NS
    )

    @functools.partial(
        pl.kernel,
        out_type=jax.ShapeDtypeStruct((NC, N_PAD, DH), jnp.float32),
        mesh=mesh,
        compiler_params=pltpu.CompilerParams(use_tc_tiling_on_sc=False),
        scratch_types=[
            pltpu.VMEM((B_IDX, CHUNK), jnp.int32),
            pltpu.VMEM((B_IDX, CHUNK), jnp.int32),
            [pltpu.VMEM((CHUNK, DH), jnp.float32) for _ in range(NBUF)],
            pltpu.MemorySpace.VMEM_SHARED((N, DH), jnp.float32),
            pltpu.MemorySpace.VMEM_SHARED((N_PAD, DH), jnp.float32),
            [pltpu.SemaphoreType.DMA for _ in range(NBUF)],
            [pltpu.SemaphoreType.DMA for _ in range(NBUF)],
        ],
    )
    def sc_kernel(xa_hbm, src_hbm, dst_hbm, z_hbm, out_hbm,
                  idx_s, idx_d, rows, x_cache, agg_sh, gsem, ssem):
        c = lax.axis_index("c")
        s = lax.axis_index("s")

        @pl.when(s < NS - 1)
        def _():
            pltpu.sync_copy(xa_hbm.at[c, pl.ds(s * CROWS, CROWS)],
                            x_cache.at[pl.ds(s * CROWS, CROWS)])

        @pl.when(s == NS - 1)
        def _():
            pltpu.sync_copy(xa_hbm.at[c, pl.ds((NS - 1) * CROWS, N - (NS - 1) * CROWS)],
                            x_cache.at[pl.ds((NS - 1) * CROWS, N - (NS - 1) * CROWS)])

        pltpu.sync_copy(z_hbm, rows[0])
        for k in range(WCHUNKS):
            pltpu.sync_copy(rows[0], agg_sh.at[pl.ds(s * ROWS_PER_TILE + k * CHUNK, CHUNK)])
        plsc.subcore_barrier()

        def blk_body(gblk, carry):
            pltpu.sync_copy(src_hbm.at[s, gblk], idx_s)
            pltpu.sync_copy(dst_hbm.at[s, gblk], idx_d)

            def body(g, carry2):
                base = g * NBUF
                descs = []
                for r in range(NBUF):
                    descs.append(pltpu.async_copy(
                        x_cache.at[idx_s.at[base + r]], rows[r], gsem[r]))
                sdescs = []
                for r in range(NBUF):
                    descs[r].wait()
                    sdescs.append(pltpu.async_copy(
                        rows[r], agg_sh.at[idx_d.at[base + r]], ssem[r], add=True))
                for r in range(NBUF):
                    sdescs[r].wait()
                return carry2

            lax.fori_loop(0, B_IDX // NBUF, body, 0)
            return carry

        lax.fori_loop(0, NBLK, blk_body, 0)
        plsc.subcore_barrier()

        for k in range(WCHUNKS):
            off = s * ROWS_PER_TILE + k * CHUNK
            pltpu.sync_copy(agg_sh.at[pl.ds(off, CHUNK)], rows[k % NBUF])
            pltpu.sync_copy(rows[k % NBUF], out_hbm.at[c, pl.ds(off, CHUNK)])

    return sc_kernel(xas, srcs, dsts, zrows)


BLK = 1000


def _tc_body(p_ref, xa_ref, wl_ref, wc_ref, wf_ref, b1_ref, bf_ref, o_ref):
    xa = xa_ref[...]
    wl = wl_ref[...]
    t = (jnp.dot(p_ref[0], wl[:DH, :], preferred_element_type=jnp.float32)
         + jnp.dot(p_ref[1], wl[DH:, :], preferred_element_type=jnp.float32)
         + jnp.dot(xa, wc_ref[...], preferred_element_type=jnp.float32)
         + b1_ref[...])
    o_ref[...] = (jnp.dot(jnp.maximum(t, 0.0), wf_ref[...],
                          preferred_element_type=jnp.float32)
                  + bf_ref[...])


def _tc_fused(partials, x_a, wl_t, wc_t, wf_t, b1, bf):
    return pl.pallas_call(
        _tc_body,
        grid=(N // BLK,),
        in_specs=[
            pl.BlockSpec((NC, BLK, DH), lambda j: (0, j, 0)),
            pl.BlockSpec((BLK, D), lambda j: (j, 0)),
            pl.BlockSpec((D, D), lambda j: (0, 0)),
            pl.BlockSpec((D, D), lambda j: (0, 0)),
            pl.BlockSpec((D, D), lambda j: (0, 0)),
            pl.BlockSpec((1, D), lambda j: (0, 0)),
            pl.BlockSpec((1, D), lambda j: (0, 0)),
        ],
        out_specs=pl.BlockSpec((BLK, D), lambda j: (j, 0)),
        out_shape=jax.ShapeDtypeStruct((N, D), jnp.float32),
    )(partials, x_a, wl_t, wc_t, wf_t, b1, bf)


def kernel(x_A, x_B, edge_r0, edge_r1,
           Wl0, bl0, W00, b00, W10, b10,
           Wl1, bl1, W01, b01, W11, b11,
           Wm0, bm0, Wm1, bm1, Wout, bout):
    src = edge_r1[1]
    dst = edge_r1[0]
    pad = E_PAD - E
    srcs = jnp.concatenate([src, jnp.zeros((pad,), jnp.int32)])
    pad_dst = N + (jnp.arange(pad, dtype=jnp.int32) % (N_PAD - N))
    dsts = jnp.concatenate([dst, pad_dst])
    srcs = srcs.reshape(NS, NBLK, B_IDX, CHUNK)
    dsts = dsts.reshape(NS, NBLK, B_IDX, CHUNK)
    zrows = jnp.zeros((CHUNK, DH), jnp.float32)
    xas = jnp.stack([x_A[:, :DH], x_A[:, DH:]], axis=0)

    partials = _sc_segment_sum(xas, srcs, dsts, zrows)

    eye = jnp.eye(D, dtype=jnp.float32)
    wl_t = Wl1.T
    wc_t = (W01 + W11).T + eye
    b1 = (bl1 + b01 + b11).reshape(1, D)
    wf_t = (Wout @ Wm1).T
    bf = (bm1 @ Wout.T + bout).reshape(1, D)

    return _tc_fused(partials, x_A, wl_t, wc_t, wf_t, b1, bf)


# speedup vs baseline: 1.0292x; 1.0015x over previous
import functools

import jax
import jax.numpy as jnp
from jax import lax
from jax.experimental import pallas as pl
from jax.experimental.pallas import tpu as pltpu
from jax.experimental.pallas import tpu_sc as plsc

N = 10000
D = 128
DH = D // 2
E = 320000

NC = 2
NS = 16
CHUNK = 128
C_PER_T = 160
B_IDX = 40
NBLK = C_PER_T // B_IDX
E_PAD = NS * C_PER_T * CHUNK
N_PAD = 10240
ROWS_PER_TILE = N_PAD // NS
WCHUNKS = ROWS_PER_TILE // CHUNK
NBUF = 4
CROWS = 632


def _sc_segment_sum(xas, srcs, dsts, zrows):
    mesh = plsc.VectorSubcoreMesh(
        core_axis_name="c", subcore_axis_name="s", num_cores=NC, num_subcores=NS
    )

    @functools.partial(
        pl.kernel,
        out_type=jax.ShapeDtypeStruct((NC, N_PAD, DH), jnp.float32),
        mesh=mesh,
        compiler_params=pltpu.CompilerParams(use_tc_tiling_on_sc=False),
        scratch_types=[
            pltpu.VMEM((B_IDX, CHUNK), jnp.int32),
            pltpu.VMEM((B_IDX, CHUNK), jnp.int32),
            [pltpu.VMEM((CHUNK, DH), jnp.float32) for _ in range(NBUF)],
            pltpu.MemorySpace.VMEM_SHARED((N, DH), jnp.float32),
            pltpu.MemorySpace.VMEM_SHARED((N_PAD, DH), jnp.float32),
            [pltpu.SemaphoreType.DMA for _ in range(NBUF)],
            [pltpu.SemaphoreType.DMA for _ in range(NBUF)],
        ],
    )
    def sc_kernel(xa_hbm, src_hbm, dst_hbm, z_hbm, out_hbm,
                  idx_s, idx_d, rows, x_cache, agg_sh, gsem, ssem):
        c = lax.axis_index("c")
        s = lax.axis_index("s")

        @pl.when(s < NS - 1)
        def _():
            pltpu.sync_copy(xa_hbm.at[c, pl.ds(s * CROWS, CROWS)],
                            x_cache.at[pl.ds(s * CROWS, CROWS)])

        @pl.when(s == NS - 1)
        def _():
            pltpu.sync_copy(xa_hbm.at[c, pl.ds((NS - 1) * CROWS, N - (NS - 1) * CROWS)],
                            x_cache.at[pl.ds((NS - 1) * CROWS, N - (NS - 1) * CROWS)])

        pltpu.sync_copy(z_hbm, rows[0])
        for k in range(WCHUNKS):
            pltpu.sync_copy(rows[0], agg_sh.at[pl.ds(s * ROWS_PER_TILE + k * CHUNK, CHUNK)])
        plsc.subcore_barrier()

        def blk_body(gblk, carry):
            pltpu.sync_copy(src_hbm.at[s, gblk], idx_s)
            pltpu.sync_copy(dst_hbm.at[s, gblk], idx_d)

            def body(g, carry2):
                base = g * NBUF
                descs = []
                for r in range(NBUF):
                    descs.append(pltpu.async_copy(
                        x_cache.at[idx_s.at[base + r]], rows[r], gsem[r]))
                sdescs = []
                for r in range(NBUF):
                    descs[r].wait()
                    sdescs.append(pltpu.async_copy(
                        rows[r], agg_sh.at[idx_d.at[base + r]], ssem[r], add=True))
                for r in range(NBUF):
                    sdescs[r].wait()
                return carry2

            lax.fori_loop(0, B_IDX // NBUF, body, 0)
            return carry

        lax.fori_loop(0, NBLK, blk_body, 0)
        plsc.subcore_barrier()

        for k in range(WCHUNKS):
            off = s * ROWS_PER_TILE + k * CHUNK
            pltpu.sync_copy(agg_sh.at[pl.ds(off, CHUNK)], rows[k % NBUF])
            pltpu.sync_copy(rows[k % NBUF], out_hbm.at[c, pl.ds(off, CHUNK)])

    return sc_kernel(xas, srcs, dsts, zrows)


BLK = 1000


def _tc_body(p_ref, xa_ref, wl_ref, wc_ref, wf_ref, b1_ref, bf_ref, o_ref):
    xa = xa_ref[...]
    wl = wl_ref[...]
    t = (jnp.dot(p_ref[0], wl[:DH, :], preferred_element_type=jnp.float32)
         + jnp.dot(p_ref[1], wl[DH:, :], preferred_element_type=jnp.float32)
         + jnp.dot(xa, wc_ref[...], preferred_element_type=jnp.float32)
         + b1_ref[...])
    o_ref[...] = (jnp.dot(jnp.maximum(t, 0.0), wf_ref[...],
                          preferred_element_type=jnp.float32)
                  + bf_ref[...])


def _tc_fused(partials, x_a, wl_t, wc_t, wf_t, b1, bf):
    return pl.pallas_call(
        _tc_body,
        grid=(N // BLK,),
        in_specs=[
            pl.BlockSpec((NC, BLK, DH), lambda j: (0, j, 0)),
            pl.BlockSpec((BLK, D), lambda j: (j, 0)),
            pl.BlockSpec((D, D), lambda j: (0, 0)),
            pl.BlockSpec((D, D), lambda j: (0, 0)),
            pl.BlockSpec((D, D), lambda j: (0, 0)),
            pl.BlockSpec((1, D), lambda j: (0, 0)),
            pl.BlockSpec((1, D), lambda j: (0, 0)),
        ],
        out_specs=pl.BlockSpec((BLK, D), lambda j: (j, 0)),
        out_shape=jax.ShapeDtypeStruct((N, D), jnp.float32),
    )(partials, x_a, wl_t, wc_t, wf_t, b1, bf)


def kernel(x_A, x_B, edge_r0, edge_r1,
           Wl0, bl0, W00, b00, W10, b10,
           Wl1, bl1, W01, b01, W11, b11,
           Wm0, bm0, Wm1, bm1, Wout, bout):
    src = edge_r1[1]
    dst = edge_r1[0]
    pad = E_PAD - E
    srcs = jnp.concatenate([src, jnp.zeros((pad,), jnp.int32)])
    pad_dst = N + (jnp.arange(pad, dtype=jnp.int32) % (N_PAD - N))
    dsts = jnp.concatenate([dst, pad_dst])
    srcs = srcs.reshape(NS, NBLK, B_IDX, CHUNK)
    dsts = dsts.reshape(NS, NBLK, B_IDX, CHUNK)
    zrows = jnp.zeros((CHUNK, DH), jnp.float32)
    xas = jnp.stack([x_A[:, :DH], x_A[:, DH:]], axis=0)

    partials = _sc_segment_sum(xas, srcs, dsts, zrows)

    eye = jnp.eye(D, dtype=jnp.float32)
    wl_t = Wl1.T
    wc_t = (W01 + W11).T + eye
    b1 = (bl1 + b01 + b11).reshape(1, D)
    wf_t = (Wout @ Wm1).T
    bf = (bm1 @ Wout.T + bout).reshape(1, D)

    return _tc_fused(partials, x_A, wl_t, wc_t, wf_t, b1, bf)


# speedup vs baseline: 1.1145x; 1.0829x over previous
import functools

import jax
import jax.numpy as jnp
from jax import lax
from jax.experimental import pallas as pl
from jax.experimental.pallas import tpu as pltpu
from jax.experimental.pallas import tpu_sc as plsc

N = 10000
D = 128
DH = D // 2
E = 320000

NC = 2
NS = 16
CHUNK = 128
C_PER_T = 160
B_IDX = 40
NBLK = C_PER_T // B_IDX
E_PAD = NS * C_PER_T * CHUNK
N_PAD = 10240
ROWS_PER_TILE = N_PAD // NS
WCHUNKS = ROWS_PER_TILE // CHUNK
NBUF = 4
CROWS = 632


def _sc_segment_sum(x_a, srcs, dsts, zrows):
    mesh = plsc.VectorSubcoreMesh(
        core_axis_name="c", subcore_axis_name="s", num_cores=NC, num_subcores=NS
    )

    @functools.partial(
        pl.kernel,
        out_type=jax.ShapeDtypeStruct((NC, N_PAD, DH), jnp.float32),
        mesh=mesh,
        compiler_params=pltpu.CompilerParams(use_tc_tiling_on_sc=False),
        scratch_types=[
            pltpu.VMEM((B_IDX, CHUNK), jnp.int32),
            pltpu.VMEM((B_IDX, CHUNK), jnp.int32),
            [pltpu.VMEM((CHUNK, DH), jnp.float32) for _ in range(NBUF)],
            pltpu.MemorySpace.VMEM_SHARED((N, DH), jnp.float32),
            pltpu.MemorySpace.VMEM_SHARED((N_PAD, DH), jnp.float32),
            [pltpu.SemaphoreType.DMA for _ in range(NBUF)],
            [pltpu.SemaphoreType.DMA for _ in range(NBUF)],
        ],
    )
    def sc_kernel(xa_hbm, src_hbm, dst_hbm, z_hbm, out_hbm,
                  idx_s, idx_d, rows, x_cache, agg_sh, gsem, ssem):
        c = lax.axis_index("c")
        s = lax.axis_index("s")

        @pl.when(s < NS - 1)
        def _():
            pltpu.sync_copy(xa_hbm.at[pl.ds(s * CROWS, CROWS), pl.ds(c * DH, DH)],
                            x_cache.at[pl.ds(s * CROWS, CROWS)])

        @pl.when(s == NS - 1)
        def _():
            pltpu.sync_copy(xa_hbm.at[pl.ds((NS - 1) * CROWS, N - (NS - 1) * CROWS),
                                      pl.ds(c * DH, DH)],
                            x_cache.at[pl.ds((NS - 1) * CROWS, N - (NS - 1) * CROWS)])

        pltpu.sync_copy(z_hbm, rows[0])
        for k in range(WCHUNKS):
            pltpu.sync_copy(rows[0], agg_sh.at[pl.ds(s * ROWS_PER_TILE + k * CHUNK, CHUNK)])
        plsc.subcore_barrier()

        def blk_body(gblk, carry):
            pltpu.sync_copy(src_hbm.at[s, gblk], idx_s)
            pltpu.sync_copy(dst_hbm.at[s, gblk], idx_d)

            def body(g, carry2):
                base = g * NBUF
                descs = []
                for r in range(NBUF):
                    descs.append(pltpu.async_copy(
                        x_cache.at[idx_s.at[base + r]], rows[r], gsem[r]))
                sdescs = []
                for r in range(NBUF):
                    descs[r].wait()
                    sdescs.append(pltpu.async_copy(
                        rows[r], agg_sh.at[idx_d.at[base + r]], ssem[r], add=True))
                for r in range(NBUF):
                    sdescs[r].wait()
                return carry2

            lax.fori_loop(0, B_IDX // NBUF, body, 0)
            return carry

        lax.fori_loop(0, NBLK, blk_body, 0)
        plsc.subcore_barrier()

        for k in range(WCHUNKS):
            off = s * ROWS_PER_TILE + k * CHUNK
            pltpu.sync_copy(agg_sh.at[pl.ds(off, CHUNK)], rows[k % NBUF])
            pltpu.sync_copy(rows[k % NBUF], out_hbm.at[c, pl.ds(off, CHUNK)])

    return sc_kernel(x_a, srcs, dsts, zrows)


BLK = 1000


def _tc_body(p_ref, xa_ref, wl_ref, wc_ref, wf_ref, b1_ref, bf_ref, o_ref):
    xa = xa_ref[...]
    wl = wl_ref[...]
    t = (jnp.dot(p_ref[0], wl[:DH, :], preferred_element_type=jnp.float32)
         + jnp.dot(p_ref[1], wl[DH:, :], preferred_element_type=jnp.float32)
         + jnp.dot(xa, wc_ref[...], preferred_element_type=jnp.float32)
         + b1_ref[...])
    o_ref[...] = (jnp.dot(jnp.maximum(t, 0.0), wf_ref[...],
                          preferred_element_type=jnp.float32)
                  + bf_ref[...])


def _tc_fused(partials, x_a, wl_t, wc_t, wf_t, b1, bf):
    return pl.pallas_call(
        _tc_body,
        grid=(N // BLK,),
        in_specs=[
            pl.BlockSpec((NC, BLK, DH), lambda j: (0, j, 0)),
            pl.BlockSpec((BLK, D), lambda j: (j, 0)),
            pl.BlockSpec((D, D), lambda j: (0, 0)),
            pl.BlockSpec((D, D), lambda j: (0, 0)),
            pl.BlockSpec((D, D), lambda j: (0, 0)),
            pl.BlockSpec((1, D), lambda j: (0, 0)),
            pl.BlockSpec((1, D), lambda j: (0, 0)),
        ],
        out_specs=pl.BlockSpec((BLK, D), lambda j: (j, 0)),
        out_shape=jax.ShapeDtypeStruct((N, D), jnp.float32),
    )(partials, x_a, wl_t, wc_t, wf_t, b1, bf)


def kernel(x_A, x_B, edge_r0, edge_r1,
           Wl0, bl0, W00, b00, W10, b10,
           Wl1, bl1, W01, b01, W11, b11,
           Wm0, bm0, Wm1, bm1, Wout, bout):
    src = edge_r1[1]
    dst = edge_r1[0]
    pad = E_PAD - E
    srcs = jnp.concatenate([src, jnp.zeros((pad,), jnp.int32)])
    pad_dst = N + (jnp.arange(pad, dtype=jnp.int32) % (N_PAD - N))
    dsts = jnp.concatenate([dst, pad_dst])
    srcs = srcs.reshape(NS, NBLK, B_IDX, CHUNK)
    dsts = dsts.reshape(NS, NBLK, B_IDX, CHUNK)
    zrows = jnp.zeros((CHUNK, DH), jnp.float32)

    partials = _sc_segment_sum(x_A, srcs, dsts, zrows)

    eye = jnp.eye(D, dtype=jnp.float32)
    wl_t = Wl1.T
    wc_t = (W01 + W11).T + eye
    b1 = (bl1 + b01 + b11).reshape(1, D)
    wf_t = (Wout @ Wm1).T
    bf = (bm1 @ Wout.T + bout).reshape(1, D)

    return _tc_fused(partials, x_A, wl_t, wc_t, wf_t, b1, bf)
